# baseline (speedup 1.0000x reference)
import functools

import jax
import jax.numpy as jnp
from jax import lax
from jax.experimental import pallas as pl
from jax.experimental.pallas import tpu as pltpu
from jax.experimental.pallas import tpu_sc as plsc

DICT_SIZE = 1000
_NC = 2
_NS = 16
_R = 64


def _sc_onehot_body(n_per_w, idx_hbm, out_hbm, idx_v, buf, sem):
    D = DICT_SIZE
    wid = lax.axis_index("s") * _NC + lax.axis_index("c")
    base = wid * n_per_w

    pltpu.sync_copy(idx_hbm.at[pl.ds(base * 1, n_per_w)], idx_v)

    zeros16 = jnp.zeros((16,), jnp.float32)
    ones16 = jnp.ones((16,), jnp.float32)
    lane = lax.iota(jnp.int32, 16)

    def _zero(i, carry):
        buf[pl.ds(i * 16, 16)] = zeros16
        return carry

    lax.fori_loop(0, (_R * D) // 16, _zero, 0, unroll=8)

    nb = n_per_w // _R

    def _batch(bi, carry):
        row0 = bi * _R
        for ck in range(_R // 16):
            idxs = idx_v[pl.ds(row0 + ck * 16, 16)]
            offs = (lane + ck * 16) * D + idxs
            plsc.store_scatter(buf, [offs], ones16)
        pltpu.sync_copy(buf, out_hbm.at[pl.ds((base + row0) * D, _R * D)])
        for ck in range(_R // 16):
            idxs = idx_v[pl.ds(row0 + ck * 16, 16)]
            offs = (lane + ck * 16) * D + idxs
            plsc.store_scatter(buf, [offs], zeros16)
        return carry

    lax.fori_loop(0, nb, _batch, 0)


def kernel(input, teacher_forcing):
    S, B = input.shape
    N = S * B
    n_per_w = N // (_NC * _NS)
    flat_idx = input.reshape(N).astype(jnp.int32)

    sc_call = pl.kernel(
        functools.partial(_sc_onehot_body, n_per_w),
        out_type=jax.ShapeDtypeStruct((N * DICT_SIZE,), jnp.float32),
        mesh=plsc.VectorSubcoreMesh(core_axis_name="c", subcore_axis_name="s"),
        scratch_types=[
            pltpu.VMEM((n_per_w,), jnp.int32),
            pltpu.VMEM((_R * DICT_SIZE,), jnp.float32),
            pltpu.SemaphoreType.DMA,
        ],
        compiler_params=pltpu.CompilerParams(needs_layout_passes=False),
    )
    p2 = sc_call(flat_idx).reshape(S, B, DICT_SIZE)
    return (p2, input)
